# baseline (speedup 1.0000x reference)
import jax
import jax.numpy as jnp
import numpy as np
from jax.experimental import pallas as pl

_NB = 6
_NPTS = 64
_ROWS = 4096
_COLS = 6144
_SYM = _COLS // _NB
_TILE_IN = 128 * _NB
_BLOCK_R = 512


def _weight_tile() -> np.ndarray:
    w = np.zeros((_TILE_IN, 128), np.float32)
    for s in range(128):
        for k in range(_NB):
            w[s * _NB + k, s] = float(2 ** (_NB - 1 - k))
    return w


def _body(bits_ref, w_ref, pre_ref, pim_ref, ore_ref, oim_ref):
    w = w_ref[...]
    pre = jnp.broadcast_to(pre_ref[...], (_BLOCK_R, _NPTS))
    pim = jnp.broadcast_to(pim_ref[...], (_BLOCK_R, _NPTS))
    for t in range(_SYM // 128):
        seg = bits_ref[:, t * _TILE_IN:(t + 1) * _TILE_IN].astype(jnp.bfloat16)
        idxf = jnp.dot(seg, w, preferred_element_type=jnp.float32)
        idx = idxf.astype(jnp.int32)
        ore_ref[:, t * 128:(t + 1) * 128] = jnp.take_along_axis(
            pre, idx, axis=1, mode="promise_in_bounds")
        oim_ref[:, t * 128:(t + 1) * 128] = jnp.take_along_axis(
            pim, idx, axis=1, mode="promise_in_bounds")


@jax.jit
def kernel(inputs, points):
    pre = jnp.real(points).astype(jnp.float32)
    pim = jnp.imag(points).astype(jnp.float32)
    w = jnp.asarray(_weight_tile(), dtype=jnp.bfloat16)
    grid = (_ROWS // _BLOCK_R,)
    out_shape = [
        jax.ShapeDtypeStruct((_ROWS, _SYM), jnp.float32),
        jax.ShapeDtypeStruct((_ROWS, _SYM), jnp.float32),
    ]
    ore, oim = pl.pallas_call(
        _body,
        grid=grid,
        in_specs=[
            pl.BlockSpec((_BLOCK_R, _COLS), lambda i: (i, 0)),
            pl.BlockSpec((_TILE_IN, 128), lambda i: (0, 0)),
            pl.BlockSpec((_NPTS,), lambda i: (0,)),
            pl.BlockSpec((_NPTS,), lambda i: (0,)),
        ],
        out_specs=[
            pl.BlockSpec((_BLOCK_R, _SYM), lambda i: (i, 0)),
            pl.BlockSpec((_BLOCK_R, _SYM), lambda i: (i, 0)),
        ],
        out_shape=out_shape,
    )(inputs, w, pre, pim)
    return jax.lax.complex(ore, oim)
